# baseline (speedup 1.0000x reference)
import jax
import jax.numpy as jnp
from jax import lax
from jax.experimental import pallas as pl
from jax.experimental.pallas import tpu as pltpu
from jax.experimental.pallas import tpu_sc as plsc

N = 10000
D = 128
E = 320000
NC = 2
NS = 16
NW = NC * NS
K = 16
CH = E // K
TPT = CH // NW
RING = TPT - 1
RPT = N // NS
NBUF = 8
GAH = 4


def _sc_body(x, ei3, zeros, out, acc,
             rows0, rows1, rows2, rows3, rows4, rows5, rows6, rows7,
             src_t, dst_t,
             sg0, sg1, sg2, sg3, sg4, sg5, sg6, sg7,
             ss0, ss1, ss2, ss3, ss4, ss5, ss6, ss7):
    c = lax.axis_index("c")
    s = lax.axis_index("s")
    wid = s * NC + c
    rows = (rows0, rows1, rows2, rows3, rows4, rows5, rows6, rows7)
    sg = (sg0, sg1, sg2, sg3, sg4, sg5, sg6, sg7)
    ss = (ss0, ss1, ss2, ss3, ss4, ss5, ss6, ss7)

    pltpu.sync_copy(zeros.at[pl.ds(s * RPT, RPT)], acc.at[pl.ds(s * RPT, RPT)])

    base = wid * TPT
    pltpu.sync_copy(ei3.at[0].at[pl.ds(base, TPT)], src_t)
    pltpu.sync_copy(ei3.at[1].at[pl.ds(base, TPT)], dst_t)
    plsc.subcore_barrier()

    def issue_gather(slot, j):
        pltpu.async_copy(x.at[src_t.at[j]], rows[slot], sg[slot])

    def wait_gather(slot):
        pltpu.make_async_copy(x.at[src_t.at[0]], rows[slot], sg[slot]).wait()

    def issue_scatter(slot, j):
        pltpu.async_copy(rows[slot], acc.at[dst_t.at[j]], ss[slot], add=True)

    def wait_scatter(slot):
        pltpu.make_async_copy(rows[slot], acc.at[dst_t.at[0]], ss[slot]).wait()

    for b in range(GAH):
        issue_gather(b, b)

    for b in range(NBUF):
        wait_gather(b)
        issue_scatter(b, b)
        nslot = (b + GAH) % NBUF
        if b >= NBUF - GAH:
            wait_scatter(nslot)
        issue_gather(nslot, b + GAH)

    def step(i, carry):
        j = i * NBUF
        for b in range(NBUF):
            wait_gather(b)
            issue_scatter(b, j + b)
            nslot = (b + GAH) % NBUF
            wait_scatter(nslot)
            jn = jnp.minimum(j + b + GAH, RING - 1)
            issue_gather(nslot, jn)
        return carry

    lax.fori_loop(1, RING // NBUF, step, 0)
    for b in range(GAH):
        wait_gather((RING + b) % NBUF)
    for i in range(NBUF - GAH):
        wait_scatter((RING - (NBUF - GAH) + i) % NBUF)

    pltpu.async_copy(x.at[src_t.at[RING]], rows0, sg0).wait()
    pltpu.sync_copy(rows0, acc.at[dst_t.at[RING]], add=True)

    plsc.subcore_barrier()

    pltpu.sync_copy(acc.at[pl.ds(s * RPT, RPT)], out.at[c].at[pl.ds(s * RPT, RPT)])


@jax.jit
def _sc_accumulate(x, ei3, zeros):
    mesh = plsc.VectorSubcoreMesh(core_axis_name="c", subcore_axis_name="s")
    return pl.kernel(
        _sc_body,
        out_type=jax.ShapeDtypeStruct((NC, N, D), jnp.float32),
        mesh=mesh,
        scratch_types=(
            [pltpu.VMEM_SHARED((N, D), jnp.float32)]
            + [pltpu.VMEM((K, D), jnp.float32) for _ in range(NBUF)]
            + [pltpu.VMEM((TPT, K), jnp.int32) for _ in range(2)]
            + [pltpu.SemaphoreType.DMA for _ in range(2 * NBUF)]
        ),
        compiler_params=pltpu.CompilerParams(use_tc_tiling_on_sc=False),
    )(x, ei3, zeros)


def _tc_body(p_ref, o_ref):
    p = p_ref[...]
    ssum = p[0] + p[1]
    nrm = jnp.sqrt(jnp.sum(ssum * ssum, axis=1, keepdims=True))
    o_ref[...] = jnp.maximum(2.0 * ssum / jnp.maximum(nrm, 1e-12), 0.0)


@jax.jit
def _tc_normalize(parts):
    R = 1000
    return pl.pallas_call(
        _tc_body,
        grid=(N // R,),
        in_specs=[pl.BlockSpec((NC, R, D), lambda i: (0, i, 0))],
        out_specs=pl.BlockSpec((R, D), lambda i: (i, 0)),
        out_shape=jax.ShapeDtypeStruct((N, D), jnp.float32),
    )(parts)


def kernel(x, edge_index, edge_weights, W_f, b_f, W_b, b_b):
    ei3 = edge_index.reshape(2, CH, K)
    zeros = jnp.zeros((N, D), jnp.float32)
    parts = _sc_accumulate(x, ei3, zeros)
    return _tc_normalize(parts)


# speedup vs baseline: 1.1668x; 1.1668x over previous
import jax
import jax.numpy as jnp
from jax import lax
from jax.experimental import pallas as pl
from jax.experimental.pallas import tpu as pltpu
from jax.experimental.pallas import tpu_sc as plsc

N = 10000
D = 128
E = 320000
NC = 2
NS = 16
NW = NC * NS
K = 32
CH = E // K
CPT = CH // NW
XTRA = CH - NW * CPT
RPT = N // NS
NBUF = 8
GAH = 4
PH_A = 160
PH_B = CPT - PH_A
IDXR = PH_A + 1


def _sc_body(x, ei3, zeros, out, acc,
             rows0, rows1, rows2, rows3, rows4, rows5, rows6, rows7,
             src_t, dst_t,
             sg0, sg1, sg2, sg3, sg4, sg5, sg6, sg7,
             ss0, ss1, ss2, ss3, ss4, ss5, ss6, ss7):
    c = lax.axis_index("c")
    s = lax.axis_index("s")
    wid = s * NC + c
    hi = wid < XTRA
    rows = (rows0, rows1, rows2, rows3, rows4, rows5, rows6, rows7)
    sg = (sg0, sg1, sg2, sg3, sg4, sg5, sg6, sg7)
    ss = (ss0, ss1, ss2, ss3, ss4, ss5, ss6, ss7)

    pltpu.sync_copy(zeros.at[pl.ds(s * RPT, RPT)], acc.at[pl.ds(s * RPT, RPT)])

    base = wid * CPT + jnp.minimum(wid, XTRA)

    def issue_gather(slot, j):
        pltpu.async_copy(x.at[src_t.at[j]], rows[slot], sg[slot])

    def wait_gather(slot):
        pltpu.make_async_copy(x.at[src_t.at[0]], rows[slot], sg[slot]).wait()

    def issue_scatter(slot, j):
        pltpu.async_copy(rows[slot], acc.at[dst_t.at[j]], ss[slot], add=True)

    def wait_scatter(slot):
        pltpu.make_async_copy(rows[slot], acc.at[dst_t.at[0]], ss[slot]).wait()

    def ring_phase(count):
        for b in range(GAH):
            issue_gather(b, b)
        for b in range(NBUF):
            wait_gather(b)
            issue_scatter(b, b)
            nslot = (b + GAH) % NBUF
            if b >= NBUF - GAH:
                wait_scatter(nslot)
            issue_gather(nslot, b + GAH)

        def step(i, carry):
            j = i * NBUF
            for b in range(NBUF):
                wait_gather(b)
                issue_scatter(b, j + b)
                nslot = (b + GAH) % NBUF
                wait_scatter(nslot)
                jn = jnp.minimum(j + b + GAH, count - 1)
                issue_gather(nslot, jn)
            return carry

        lax.fori_loop(1, count // NBUF, step, 0)
        for b in range(GAH):
            wait_gather((count + b) % NBUF)
        for i in range(NBUF - GAH):
            wait_scatter((count - (NBUF - GAH) + i) % NBUF)

    pltpu.sync_copy(ei3.at[0].at[pl.ds(base, PH_A)], src_t.at[pl.ds(0, PH_A)])
    pltpu.sync_copy(ei3.at[1].at[pl.ds(base, PH_A)], dst_t.at[pl.ds(0, PH_A)])
    plsc.subcore_barrier()
    ring_phase(PH_A)

    @pl.when(hi)
    def _():
        pltpu.sync_copy(ei3.at[0].at[pl.ds(base + PH_A, PH_B + 1)],
                        src_t.at[pl.ds(0, PH_B + 1)])
        pltpu.sync_copy(ei3.at[1].at[pl.ds(base + PH_A, PH_B + 1)],
                        dst_t.at[pl.ds(0, PH_B + 1)])

    @pl.when(jnp.logical_not(hi))
    def _():
        pltpu.sync_copy(ei3.at[0].at[pl.ds(base + PH_A, PH_B)],
                        src_t.at[pl.ds(0, PH_B)])
        pltpu.sync_copy(ei3.at[1].at[pl.ds(base + PH_A, PH_B)],
                        dst_t.at[pl.ds(0, PH_B)])

    ring_phase(PH_B)

    @pl.when(hi)
    def _():
        pltpu.async_copy(x.at[src_t.at[PH_B]], rows0, sg0).wait()
        pltpu.sync_copy(rows0, acc.at[dst_t.at[PH_B]], add=True)

    plsc.subcore_barrier()

    pltpu.sync_copy(acc.at[pl.ds(s * RPT, RPT)], out.at[c].at[pl.ds(s * RPT, RPT)])


@jax.jit
def _sc_accumulate(x, ei3, zeros):
    mesh = plsc.VectorSubcoreMesh(core_axis_name="c", subcore_axis_name="s")
    return pl.kernel(
        _sc_body,
        out_type=jax.ShapeDtypeStruct((NC, N, D), jnp.float32),
        mesh=mesh,
        scratch_types=(
            [pltpu.VMEM_SHARED((N, D), jnp.float32)]
            + [pltpu.VMEM((K, D), jnp.float32) for _ in range(NBUF)]
            + [pltpu.VMEM((IDXR, K), jnp.int32) for _ in range(2)]
            + [pltpu.SemaphoreType.DMA for _ in range(2 * NBUF)]
        ),
        compiler_params=pltpu.CompilerParams(use_tc_tiling_on_sc=False),
    )(x, ei3, zeros)


def _tc_body(p_ref, o_ref):
    p = p_ref[...]
    ssum = p[0] + p[1]
    nrm = jnp.sqrt(jnp.sum(ssum * ssum, axis=1, keepdims=True))
    o_ref[...] = jnp.maximum(2.0 * ssum / jnp.maximum(nrm, 1e-12), 0.0)


@jax.jit
def _tc_normalize(parts):
    R = 1000
    return pl.pallas_call(
        _tc_body,
        grid=(N // R,),
        in_specs=[pl.BlockSpec((NC, R, D), lambda i: (0, i, 0))],
        out_specs=pl.BlockSpec((R, D), lambda i: (i, 0)),
        out_shape=jax.ShapeDtypeStruct((N, D), jnp.float32),
    )(parts)


def kernel(x, edge_index, edge_weights, W_f, b_f, W_b, b_b):
    ei3 = edge_index.reshape(2, CH, K)
    zeros = jnp.zeros((N, D), jnp.float32)
    parts = _sc_accumulate(x, ei3, zeros)
    return _tc_normalize(parts)


# speedup vs baseline: 1.3246x; 1.1353x over previous
import jax
import jax.numpy as jnp
from jax import lax
from jax.experimental import pallas as pl
from jax.experimental.pallas import tpu as pltpu
from jax.experimental.pallas import tpu_sc as plsc

N = 10000
D = 128
E = 320000
NC = 2
NS = 16
NW = NC * NS
K = 32
CH = E // K
CPT = CH // NW
XTRA = CH - NW * CPT
RPT = N // NS
NBUF = 6
GAH = 4


def _sc_body(x, ei3, zeros, out, acc,
             rows0, rows1, rows2, rows3, rows4, rows5, src_t, dst_t,
             sg0, sg1, sg2, sg3, sg4, sg5, ss0, ss1, ss2, ss3, ss4, ss5):
    c = lax.axis_index("c")
    s = lax.axis_index("s")
    wid = s * NC + c
    hi = wid < XTRA
    rows = (rows0, rows1, rows2, rows3, rows4, rows5)
    sg = (sg0, sg1, sg2, sg3, sg4, sg5)
    ss = (ss0, ss1, ss2, ss3, ss4, ss5)

    pltpu.sync_copy(zeros, acc.at[pl.ds(s * RPT, RPT)])

    base = wid * CPT + jnp.minimum(wid, XTRA)

    @pl.when(hi)
    def _():
        pltpu.sync_copy(ei3.at[0].at[pl.ds(base, CPT + 1)], src_t)
        pltpu.sync_copy(ei3.at[1].at[pl.ds(base, CPT + 1)], dst_t)

    @pl.when(jnp.logical_not(hi))
    def _():
        pltpu.sync_copy(ei3.at[0].at[pl.ds(base, CPT)], src_t.at[pl.ds(0, CPT)])
        pltpu.sync_copy(ei3.at[1].at[pl.ds(base, CPT)], dst_t.at[pl.ds(0, CPT)])

    plsc.subcore_barrier()

    def issue_gather(slot, j):
        pltpu.async_copy(x.at[src_t.at[j]], rows[slot], sg[slot])

    def wait_gather(slot):
        pltpu.make_async_copy(x.at[src_t.at[0]], rows[slot], sg[slot]).wait()

    def issue_scatter(slot, j):
        pltpu.async_copy(rows[slot], acc.at[dst_t.at[j]], ss[slot], add=True)

    def wait_scatter(slot):
        pltpu.make_async_copy(rows[slot], acc.at[dst_t.at[0]], ss[slot]).wait()

    for b in range(GAH):
        issue_gather(b, b)

    for b in range(NBUF):
        wait_gather(b)
        issue_scatter(b, b)
        nslot = (b + GAH) % NBUF
        if b >= NBUF - GAH:
            wait_scatter(nslot)
        issue_gather(nslot, b + GAH)

    def step(i, carry):
        j = i * NBUF
        for b in range(NBUF):
            wait_gather(b)
            issue_scatter(b, j + b)
            nslot = (b + GAH) % NBUF
            wait_scatter(nslot)
            jn = jnp.minimum(j + b + GAH, CPT - 1)
            issue_gather(nslot, jn)
        return carry

    lax.fori_loop(1, CPT // NBUF, step, 0)
    for b in range(GAH):
        wait_gather((CPT + b) % NBUF)
    wait_scatter((CPT - 2) % NBUF)
    wait_scatter((CPT - 1) % NBUF)

    @pl.when(hi)
    def _():
        pltpu.async_copy(x.at[src_t.at[CPT]], rows0, sg0).wait()
        pltpu.sync_copy(rows0, acc.at[dst_t.at[CPT]], add=True)

    plsc.subcore_barrier()

    pltpu.sync_copy(acc.at[pl.ds(s * RPT, RPT)], out.at[c].at[pl.ds(s * RPT, RPT)])


@jax.jit
def _sc_accumulate(x, ei3, zeros):
    mesh = plsc.VectorSubcoreMesh(core_axis_name="c", subcore_axis_name="s")
    return pl.kernel(
        _sc_body,
        out_type=jax.ShapeDtypeStruct((NC, N, D), jnp.float32),
        mesh=mesh,
        scratch_types=(
            [pltpu.VMEM_SHARED((N, D), jnp.float32)]
            + [pltpu.VMEM((K, D), jnp.float32) for _ in range(NBUF)]
            + [pltpu.VMEM((CPT + 1, K), jnp.int32) for _ in range(2)]
            + [pltpu.SemaphoreType.DMA for _ in range(2 * NBUF)]
        ),
        compiler_params=pltpu.CompilerParams(use_tc_tiling_on_sc=False),
    )(x, ei3, zeros)


def _tc_body(p_ref, o_ref):
    p = p_ref[...]
    ssum = p[0] + p[1]
    nrm = jnp.sqrt(jnp.sum(ssum * ssum, axis=1, keepdims=True))
    o_ref[...] = jnp.maximum(2.0 * ssum / jnp.maximum(nrm, 1e-12), 0.0)


@jax.jit
def _tc_normalize(parts):
    R = 2000
    return pl.pallas_call(
        _tc_body,
        grid=(N // R,),
        in_specs=[pl.BlockSpec((NC, R, D), lambda i: (0, i, 0))],
        out_specs=pl.BlockSpec((R, D), lambda i: (i, 0)),
        out_shape=jax.ShapeDtypeStruct((N, D), jnp.float32),
    )(parts)


def kernel(x, edge_index, edge_weights, W_f, b_f, W_b, b_b):
    ei3 = edge_index.reshape(2, CH, K)
    zeros = jnp.zeros((RPT, D), jnp.float32)
    parts = _sc_accumulate(x, ei3, zeros)
    return _tc_normalize(parts)
